# baseline (speedup 1.0000x reference)
import functools

import jax
import jax.numpy as jnp
from jax import lax
from jax.experimental import pallas as pl
from jax.experimental.pallas import tpu as pltpu
from jax.experimental.pallas import tpu_sc as plsc

N = 10000
E = 320000
G = 16
H1 = 128
H2 = 64
HEADS = 8
NITER = 30

_NC = 2
_NS = 16
_MESH = plsc.VectorSubcoreMesh(core_axis_name="c", subcore_axis_name="s")

_NPAD = 10240
_RPT = _NPAD // _NS
_EPC = E // _NC
_EPT = E // (_NC * _NS)


def _wid(c, s):
    return s * _NC + c


_K1 = 2000
_NCH1 = _EPT // _K1


@functools.partial(
    pl.kernel,
    out_type=(
        jax.ShapeDtypeStruct((_NC, _NPAD), jnp.float32),
        jax.ShapeDtypeStruct((_NC, _NPAD), jnp.float32),
        jax.ShapeDtypeStruct((_NC, _NPAD), jnp.float32),
    ),
    mesh=_MESH,
    scratch_types=[
        pltpu.VMEM((_K1,), jnp.int32),
        pltpu.VMEM((_K1,), jnp.int32),
        pltpu.VMEM((_K1,), jnp.float32),
        pltpu.VMEM((_K1,), jnp.float32),
        pltpu.VMEM((_K1,), jnp.float32),
        pltpu.VMEM_SHARED((_NPAD,), jnp.float32),
        pltpu.VMEM_SHARED((_NPAD,), jnp.float32),
        pltpu.VMEM_SHARED((_NPAD,), jnp.float32),
        pltpu.SemaphoreType.DMA,
    ],
)
def _edge_init_kernel(x_hbm, row_hbm, col_hbm, zv_hbm,
                      outx_hbm, outm_hbm, outd_hbm,
                      idx_r, idx_c, xg, mm, ones, accx, accm, accd, sem):
    c = lax.axis_index("c")
    s = lax.axis_index("s")
    r0 = s * _RPT
    pltpu.sync_copy(zv_hbm.at[pl.ds(r0, _RPT)], accx.at[pl.ds(r0, _RPT)])
    pltpu.sync_copy(zv_hbm.at[pl.ds(r0, _RPT)], accm.at[pl.ds(r0, _RPT)])
    pltpu.sync_copy(zv_hbm.at[pl.ds(r0, _RPT)], accd.at[pl.ds(r0, _RPT)])
    onev = jnp.ones((16,), jnp.float32)
    for i in range(_K1 // 16):
        ones[pl.ds(i * 16, 16)] = onev
    plsc.subcore_barrier()
    base0 = c * _EPC + s * _EPT

    def body(i, carry):
        base = base0 + i * _K1
        pltpu.sync_copy(row_hbm.at[pl.ds(base, _K1)], idx_r)
        pltpu.async_copy(x_hbm.at[idx_r], xg, sem).wait()
        pltpu.sync_copy(col_hbm.at[pl.ds(base, _K1)], idx_c)

        def vb(j, carry2):
            v = xg[pl.ds(j * 16, 16)]
            mm[pl.ds(j * 16, 16)] = jnp.where(v != 0.0, 1.0, 0.0)
            return carry2

        lax.fori_loop(0, _K1 // 16, vb, 0)
        pltpu.sync_copy(xg, accx.at[idx_c], add=True)
        pltpu.sync_copy(mm, accm.at[idx_c], add=True)
        pltpu.sync_copy(ones, accd.at[idx_r], add=True)
        return carry

    lax.fori_loop(0, _NCH1, body, 0)
    plsc.subcore_barrier()
    pltpu.sync_copy(accx.at[pl.ds(r0, _RPT)], outx_hbm.at[c, pl.ds(r0, _RPT)])
    pltpu.sync_copy(accm.at[pl.ds(r0, _RPT)], outm_hbm.at[c, pl.ds(r0, _RPT)])
    pltpu.sync_copy(accd.at[pl.ds(r0, _RPT)], outd_hbm.at[c, pl.ds(r0, _RPT)])


_K2 = 200
_NCH2 = _EPT // _K2


@functools.partial(
    pl.kernel,
    out_type=(
        jax.ShapeDtypeStruct((_NC, _NPAD, H1), jnp.float32),
        jax.ShapeDtypeStruct((_NC, _NPAD), jnp.float32),
    ),
    mesh=_MESH,
    scratch_types=[
        pltpu.VMEM((_K2,), jnp.int32),
        pltpu.VMEM((_K2,), jnp.int32),
        pltpu.VMEM((_K2, H1), jnp.float32),
        pltpu.VMEM((_K2,), jnp.float32),
        pltpu.VMEM_SHARED((_NPAD, H1), jnp.float32),
        pltpu.VMEM_SHARED((_NPAD,), jnp.float32),
        pltpu.SemaphoreType.DMA,
        pltpu.SemaphoreType.DMA,
    ],
)
def _gin_mask_kernel(h_hbm, m_hbm, row_hbm, col_hbm, zm_hbm, zv_hbm,
                     outa_hbm, outm_hbm,
                     idx_r, idx_c, rows, mg, acc, accm, sem, sem2):
    c = lax.axis_index("c")
    s = lax.axis_index("s")
    r0 = s * _RPT
    pltpu.sync_copy(zm_hbm.at[pl.ds(r0, _RPT)], acc.at[pl.ds(r0, _RPT)])
    pltpu.sync_copy(zv_hbm.at[pl.ds(r0, _RPT)], accm.at[pl.ds(r0, _RPT)])
    plsc.subcore_barrier()
    base0 = c * _EPC + s * _EPT

    def body(i, carry):
        base = base0 + i * _K2
        pltpu.sync_copy(row_hbm.at[pl.ds(base, _K2)], idx_r)
        cp1 = pltpu.async_copy(h_hbm.at[idx_r], rows, sem)
        cp2 = pltpu.async_copy(m_hbm.at[idx_r], mg, sem2)
        pltpu.sync_copy(col_hbm.at[pl.ds(base, _K2)], idx_c)
        cp1.wait()
        cp2.wait()
        pltpu.sync_copy(rows, acc.at[idx_c], add=True)
        pltpu.sync_copy(mg, accm.at[idx_c], add=True)
        return carry

    lax.fori_loop(0, _NCH2, body, 0)
    plsc.subcore_barrier()
    pltpu.sync_copy(acc.at[pl.ds(r0, _RPT)], outa_hbm.at[c, pl.ds(r0, _RPT)])
    pltpu.sync_copy(accm.at[pl.ds(r0, _RPT)], outm_hbm.at[c, pl.ds(r0, _RPT)])


_K3 = 1000
_NCH3 = _EPT // _K3


@functools.partial(
    pl.kernel,
    out_type=(
        jax.ShapeDtypeStruct((E, 16), jnp.float32),
        jax.ShapeDtypeStruct((_NC, _NPAD, 16), jnp.float32),
        jax.ShapeDtypeStruct((_NC, _NPAD), jnp.float32),
    ),
    mesh=_MESH,
    compiler_params=pltpu.CompilerParams(use_tc_tiling_on_sc=False),
    scratch_types=[
        pltpu.VMEM((_K3,), jnp.int32),
        pltpu.VMEM((_K3,), jnp.int32),
        pltpu.VMEM((_K3, 16), jnp.float32),
        pltpu.VMEM((_K3, 16), jnp.float32),
        pltpu.VMEM((_K3,), jnp.float32),
        pltpu.VMEM_SHARED((_NPAD, 16), jnp.float32),
        pltpu.VMEM_SHARED((_NPAD,), jnp.float32),
        pltpu.SemaphoreType.DMA,
        pltpu.SemaphoreType.DMA,
        pltpu.SemaphoreType.DMA,
    ],
)
def _gat1_kernel(asrc_hbm, adst_hbm, m_hbm, row_hbm, col_hbm, z16_hbm, zv_hbm,
                 ea_hbm, den_hbm, outm_hbm,
                 idx_r, idx_c, ar, bc, mg, accd, accm, sem, sem2, sem3):
    c = lax.axis_index("c")
    s = lax.axis_index("s")
    r0 = s * _RPT
    pltpu.sync_copy(z16_hbm.at[pl.ds(r0, _RPT)], accd.at[pl.ds(r0, _RPT)])
    pltpu.sync_copy(zv_hbm.at[pl.ds(r0, _RPT)], accm.at[pl.ds(r0, _RPT)])
    plsc.subcore_barrier()
    base0 = c * _EPC + s * _EPT

    def body(i, carry):
        base = base0 + i * _K3
        pltpu.sync_copy(row_hbm.at[pl.ds(base, _K3)], idx_r)
        pltpu.sync_copy(col_hbm.at[pl.ds(base, _K3)], idx_c)
        cp1 = pltpu.async_copy(asrc_hbm.at[idx_r], ar, sem)
        cp2 = pltpu.async_copy(adst_hbm.at[idx_c], bc, sem2)
        cp3 = pltpu.async_copy(m_hbm.at[idx_r], mg, sem3)
        cp1.wait()
        cp2.wait()
        cp3.wait()

        def vb(e, carry2):
            z = ar[e, :] + bc[e, :]
            z = jnp.where(z > 0.0, z, z * 0.2)
            ar[e, :] = jnp.exp(z)
            return carry2

        lax.fori_loop(0, _K3, vb, 0)
        pltpu.sync_copy(ar, accd.at[idx_c], add=True)
        pltpu.sync_copy(mg, accm.at[idx_c], add=True)
        pltpu.sync_copy(ar, ea_hbm.at[pl.ds(base, _K3)])
        return carry

    lax.fori_loop(0, _NCH3, body, 0)
    plsc.subcore_barrier()
    pltpu.sync_copy(accd.at[pl.ds(r0, _RPT)], den_hbm.at[c, pl.ds(r0, _RPT)])
    pltpu.sync_copy(accm.at[pl.ds(r0, _RPT)], outm_hbm.at[c, pl.ds(r0, _RPT)])


_K4 = 200
_NCH4 = _EPT // _K4


@functools.partial(
    pl.kernel,
    out_type=jax.ShapeDtypeStruct((4, _NC, _NPAD, H1), jnp.float32),
    mesh=_MESH,
    scratch_types=[
        pltpu.VMEM((_K4,), jnp.int32),
        pltpu.VMEM((_K4,), jnp.int32),
        pltpu.VMEM((_K4, H1), jnp.float32),
        pltpu.VMEM((_K4 * 16,), jnp.float32),
        pltpu.VMEM_SHARED((_NPAD, H1), jnp.float32),
        pltpu.SemaphoreType.DMA,
    ],
)
def _gat2_kernel(hh0_hbm, hh1_hbm, hh2_hbm, hh3_hbm, eaf_hbm, row_hbm, col_hbm,
                 zm_hbm, out_hbm, idx_r, idx_c, rows2, eaf, acc, sem):
    c = lax.axis_index("c")
    s = lax.axis_index("s")
    r0 = s * _RPT
    base0 = c * _EPC + s * _EPT
    tables = (hh0_hbm, hh1_hbm, hh2_hbm, hh3_hbm)

    for cnk in range(4):
        hh_hbm = tables[cnk]
        pltpu.sync_copy(zm_hbm.at[pl.ds(r0, _RPT)], acc.at[pl.ds(r0, _RPT)])
        plsc.subcore_barrier()

        def body(i, carry):
            base = base0 + i * _K4
            pltpu.sync_copy(row_hbm.at[pl.ds(base, _K4)], idx_r)
            cp1 = pltpu.async_copy(hh_hbm.at[idx_r], rows2, sem)
            pltpu.sync_copy(eaf_hbm.at[pl.ds(base * 16, _K4 * 16)], eaf)
            pltpu.sync_copy(col_hbm.at[pl.ds(base, _K4)], idx_c)
            cp1.wait()

            def eb(e, carry2):
                ev = eaf[pl.ds(e * 16, 16)]
                for j in range(8):
                    hidx = jnp.full((16,), 2 * cnk + j // 4, jnp.int32)
                    m = ev.at[hidx].get(mode="promise_in_bounds")
                    rows2[e, pl.ds(j * 16, 16)] = (
                        rows2[e, pl.ds(j * 16, 16)] * m)
                return carry2

            lax.fori_loop(0, _K4, eb, 0)
            pltpu.sync_copy(rows2, acc.at[idx_c], add=True)
            return carry

        lax.fori_loop(0, _NCH4, body, 0)
        plsc.subcore_barrier()
        pltpu.sync_copy(acc.at[pl.ds(r0, _RPT)],
                        out_hbm.at[cnk, c, pl.ds(r0, _RPT)])


_K5 = 1000
_NCH5 = _EPT // _K5


@functools.partial(
    pl.kernel,
    out_type=jax.ShapeDtypeStruct((_NC, 16, 16), jnp.float32),
    mesh=_MESH,
    compiler_params=pltpu.CompilerParams(use_tc_tiling_on_sc=False),
    scratch_types=[
        pltpu.VMEM((_K5,), jnp.int32),
        pltpu.VMEM((_K5,), jnp.int32),
        pltpu.VMEM((_K5, 16), jnp.float32),
        pltpu.VMEM((_K5, 16), jnp.float32),
        pltpu.VMEM((_K5,), jnp.int32),
        pltpu.VMEM((16, 16), jnp.float32),
        pltpu.VMEM_SHARED((16, 16), jnp.float32),
        pltpu.SemaphoreType.DMA,
        pltpu.SemaphoreType.DMA,
        pltpu.SemaphoreType.DMA,
    ],
)
def _cut_kernel(p_hbm, b_hbm, row_hbm, col_hbm,
                out_hbm, idx_r, idx_c, pr, pc, bg, zv, acc, sem, sem2, sem3):
    c = lax.axis_index("c")
    s = lax.axis_index("s")
    zvec16 = jnp.zeros((16,), jnp.float32)
    for i in range(16):
        zv[i, :] = zvec16

    @pl.when(s == 0)
    def _():
        pltpu.sync_copy(zv, acc)

    plsc.subcore_barrier()
    base0 = c * _EPC + s * _EPT

    def body(i, carry):
        base = base0 + i * _K5
        pltpu.sync_copy(row_hbm.at[pl.ds(base, _K5)], idx_r)
        pltpu.sync_copy(col_hbm.at[pl.ds(base, _K5)], idx_c)
        cp1 = pltpu.async_copy(p_hbm.at[idx_r], pr, sem)
        cp2 = pltpu.async_copy(p_hbm.at[idx_c], pc, sem2)
        cp3 = pltpu.async_copy(b_hbm.at[idx_r], bg, sem3)
        cp1.wait()
        cp2.wait()
        cp3.wait()

        def vb(e, carry2):
            pr[e, :] = pr[e, :] * pc[e, :]
            return carry2

        lax.fori_loop(0, _K5, vb, 0)
        pltpu.sync_copy(pr, acc.at[bg], add=True)
        return carry

    lax.fori_loop(0, _NCH5, body, 0)
    plsc.subcore_barrier()

    @pl.when(s == 0)
    def _():
        pltpu.sync_copy(acc, out_hbm.at[c])


_BLK = 1024
_GRID = _NPAD // _BLK


def _vspec():
    return pl.BlockSpec((_BLK, 1), lambda i: (i, 0))


def _mspec():
    return pl.BlockSpec((_BLK, H1), lambda i: (i, 0))


def _wspec(shape):
    return pl.BlockSpec(shape, lambda i: (0,) * len(shape))


def _gin1_body(x_ref, xa0_ref, xa1_ref, mm0_ref, mm1_ref,
               W1_ref, b1_ref, W2_ref, b2_ref, g_ref, bt_ref,
               h_ref, mk_ref):
    xv = x_ref[...]
    agg = xv + xa0_ref[...] + xa1_ref[...]
    z = jnp.maximum(agg * W1_ref[...] + b1_ref[...], 0.0)
    z = jnp.maximum(jnp.dot(z, W2_ref[...], preferred_element_type=jnp.float32)
                    + b2_ref[...], 0.0)
    z = z * g_ref[...] + bt_ref[...]
    mown = jnp.where(xv != 0.0, 1.0, 0.0)
    mk = jnp.where(mown + mm0_ref[...] + mm1_ref[...] > 0.0, 1.0, 0.0)
    mk_ref[...] = mk
    h_ref[...] = z * mk


def _gin1_tc(xf, xa0, xa1, mm0, mm1, p):
    return pl.pallas_call(
        _gin1_body,
        grid=(_GRID,),
        in_specs=[_vspec(), _vspec(), _vspec(), _vspec(), _vspec(),
                  _wspec((1, H1)), _wspec((1, H1)), _wspec((H1, H1)),
                  _wspec((1, H1)), _wspec((1, H1)), _wspec((1, H1))],
        out_specs=(_mspec(), _vspec()),
        out_shape=(jax.ShapeDtypeStruct((_NPAD, H1), jnp.float32),
                   jax.ShapeDtypeStruct((_NPAD, 1), jnp.float32)),
    )(xf, xa0, xa1, mm0, mm1, p['c1_W1'], p['c1_b1'].reshape(1, H1),
      p['c1_W2'], p['c1_b2'].reshape(1, H1), p['c1_g'].reshape(1, H1),
      p['c1_bt'].reshape(1, H1))


def _gin_body(h_ref, p0_ref, p1_ref, m_ref, mm0_ref, mm1_ref,
              W1_ref, b1_ref, W2_ref, b2_ref, g_ref, bt_ref,
              bng_ref, bnb_ref, h_out_ref, mk_ref):
    hv = h_ref[...]
    agg = hv + p0_ref[...] + p1_ref[...]
    z = jnp.maximum(jnp.dot(agg, W1_ref[...], preferred_element_type=jnp.float32)
                    + b1_ref[...], 0.0)
    z = jnp.maximum(jnp.dot(z, W2_ref[...], preferred_element_type=jnp.float32)
                    + b2_ref[...], 0.0)
    gi = z * g_ref[...] + bt_ref[...]
    mk = jnp.where(m_ref[...] + mm0_ref[...] + mm1_ref[...] > 0.0, 1.0, 0.0)
    mk_ref[...] = mk
    hn = (hv + gi) * mk
    h_out_ref[...] = hn * bng_ref[...] + bnb_ref[...]


def _gin_tc(h, p0, p1, m, mm0, mm1, i, p):
    return pl.pallas_call(
        _gin_body,
        grid=(_GRID,),
        in_specs=[_mspec(), _mspec(), _mspec(), _vspec(), _vspec(), _vspec(),
                  _wspec((H1, H1)), _wspec((1, H1)), _wspec((H1, H1)),
                  _wspec((1, H1)), _wspec((1, H1)), _wspec((1, H1)),
                  _wspec((1, H1)), _wspec((1, H1))],
        out_specs=(_mspec(), _vspec()),
        out_shape=(jax.ShapeDtypeStruct((_NPAD, H1), jnp.float32),
                   jax.ShapeDtypeStruct((_NPAD, 1), jnp.float32)),
    )(h, p0, p1, m, mm0, mm1,
      p['cv%d_W1' % i], p['cv%d_b1' % i].reshape(1, H1),
      p['cv%d_W2' % i], p['cv%d_b2' % i].reshape(1, H1),
      p['cv%d_g' % i].reshape(1, H1), p['cv%d_bt' % i].reshape(1, H1),
      p['bn%d_g' % i].reshape(1, H1), p['bn%d_b' % i].reshape(1, H1))


def _gatprep_body(h_ref, W_ref, A_ref, B_ref,
                  hh0_ref, hh1_ref, hh2_ref, hh3_ref, a16_ref, b16_ref):
    hh = jnp.dot(h_ref[...], W_ref[...], preferred_element_type=jnp.float32)
    asrc = jnp.dot(hh, A_ref[...], preferred_element_type=jnp.float32)
    adst = jnp.dot(hh, B_ref[...], preferred_element_type=jnp.float32)
    a16_ref[...] = jnp.concatenate([asrc, asrc], axis=1)
    b16_ref[...] = jnp.concatenate([adst, adst], axis=1)
    hh0_ref[...] = hh[:, 0:128]
    hh1_ref[...] = hh[:, 128:256]
    hh2_ref[...] = hh[:, 256:384]
    hh3_ref[...] = hh[:, 384:512]


def _gatprep_tc(h, gat_W, Asrc, Adst):
    spec16 = pl.BlockSpec((_BLK, 16), lambda i: (i, 0))
    return pl.pallas_call(
        _gatprep_body,
        grid=(_GRID,),
        in_specs=[_mspec(), _wspec((H1, HEADS * H2)),
                  _wspec((HEADS * H2, HEADS)), _wspec((HEADS * H2, HEADS))],
        out_specs=(_mspec(), _mspec(), _mspec(), _mspec(), spec16, spec16),
        out_shape=(jax.ShapeDtypeStruct((_NPAD, H1), jnp.float32),
                   jax.ShapeDtypeStruct((_NPAD, H1), jnp.float32),
                   jax.ShapeDtypeStruct((_NPAD, H1), jnp.float32),
                   jax.ShapeDtypeStruct((_NPAD, H1), jnp.float32),
                   jax.ShapeDtypeStruct((_NPAD, 16), jnp.float32),
                   jax.ShapeDtypeStruct((_NPAD, 16), jnp.float32)),
    )(h, gat_W, Asrc, Adst)


def _gatpost_body(n0a_ref, n0b_ref, n1a_ref, n1b_ref, n2a_ref, n2b_ref,
                  n3a_ref, n3b_ref, d0_ref, d1_ref, m_ref, mm0_ref, mm1_ref,
                  E2_ref, l1W_ref, l1b_ref, bng_ref, bnb_ref, l2W_ref,
                  l2b_ref, h2_ref, mk_ref):
    den = d0_ref[...] + d1_ref[...] + 1e-16
    mk = jnp.where(m_ref[...] + mm0_ref[...] + mm1_ref[...] > 0.0, 1.0, 0.0)
    mk_ref[...] = mk
    parts = []
    nums = ((n0a_ref, n0b_ref), (n1a_ref, n1b_ref), (n2a_ref, n2b_ref),
            (n3a_ref, n3b_ref))
    for cnk in range(4):
        dpair = den[:, 2 * cnk:2 * cnk + 2]
        dexp = jnp.dot(dpair, E2_ref[...], preferred_element_type=jnp.float32)
        num = nums[cnk][0][...] + nums[cnk][1][...]
        parts.append(num / dexp)
    hcat = jnp.concatenate(parts, axis=1)
    hcat = hcat * mk
    z = jnp.dot(hcat, l1W_ref[...], preferred_element_type=jnp.float32) \
        + l1b_ref[...]
    z = jnp.where(z > 0.0, z, 0.01 * z)
    z = z * mk
    z = z * bng_ref[...] + bnb_ref[...]
    z2 = jnp.dot(z, l2W_ref[...], preferred_element_type=jnp.float32) \
        + l2b_ref[0, 0]
    z2 = jnp.where(z2 > 0.0, z2, 0.01 * z2)
    h2_ref[...] = z2 * mk


def _gatpost_tc(nums8, d0, d1, m, mm0, mm1, E2, p):
    spec16 = pl.BlockSpec((_BLK, 16), lambda i: (i, 0))
    return pl.pallas_call(
        _gatpost_body,
        grid=(_GRID,),
        in_specs=[_mspec()] * 8 + [spec16, spec16, _vspec(), _vspec(),
                                   _vspec(), _wspec((2, H1)),
                                   _wspec((HEADS * H2, H1)), _wspec((1, H1)),
                                   _wspec((1, H1)), _wspec((1, H1)),
                                   _wspec((H1, 1)), _wspec((1, 1))],
        out_specs=(_vspec(), _vspec()),
        out_shape=(jax.ShapeDtypeStruct((_NPAD, 1), jnp.float32),
                   jax.ShapeDtypeStruct((_NPAD, 1), jnp.float32)),
    )(*nums8, d0, d1, m, mm0, mm1, E2,
      p['l1_W'], p['l1_b'].reshape(1, H1), p['bn2_g'].reshape(1, H1),
      p['bn2_b'].reshape(1, H1), p['l2_W'], p['l2_b'].reshape(1, 1))


def _niter_body(h_ref, mask_ref, xinit_ref, deg_ref, oh_ref, oht_ref,
                tvol_ref, probs_ref, cut1_ref):
    hv = h_ref[...]
    mv = mask_ref[...]
    xv = xinit_ref[...]
    dv = deg_ref[...]
    oh = oh_ref[...]
    oht = oht_ref[...]
    neg = jnp.float32(-jnp.inf)
    bmaxn = jnp.zeros_like(hv)
    bminn = jnp.zeros_like(hv)
    for g in range(G):
        sel = oht[g:g + 1, :]
        mg = jnp.max(jnp.where(sel > 0.0, hv, neg))
        mg = jnp.where(jnp.isfinite(mg), mg, 0.0)
        ng = -jnp.max(jnp.where(sel > 0.0, -hv, neg))
        ng = jnp.where(jnp.isfinite(ng), ng, 0.0)
        bmaxn = bmaxn + mg * sel
        bminn = bminn + ng * sel
    hv = (hv - bminn) / (bmaxn + 1e-06 - bminn)
    hv = hv * mv + mv * 1e-06 + xv
    totalvol = jnp.dot(dv, oh, preferred_element_type=jnp.float32) + 1e-06
    target = tvol_ref[...] * totalvol

    def it(i, a):
        an = jnp.dot(a, oht, preferred_element_type=jnp.float32)
        keep = (an * hv < 1.0).astype(jnp.float32)
        km = keep * mv
        xk = hv * km
        dk = dv * km
        dnk = dv * (1.0 - keep) * mv
        diff = target - jnp.dot(dnk, oh, preferred_element_type=jnp.float32)
        dot = jnp.dot(xk * dk, oh, preferred_element_type=jnp.float32)
        return diff / (dot + 1e-05)

    a = lax.fori_loop(0, NITER, it, jnp.ones((1, G), jnp.float32))
    an = jnp.dot(a, oht, preferred_element_type=jnp.float32)
    probs = jnp.clip(an * hv * mv, 0.0, 1.0)
    probs_ref[...] = probs
    cut1_ref[...] = jnp.dot(probs * dv, oh, preferred_element_type=jnp.float32)


def _niter_tc(h2, maskf, xinitf, degf, onehot, onehotT, tvol2):
    return pl.pallas_call(
        _niter_body,
        out_shape=(
            jax.ShapeDtypeStruct((1, _NPAD), jnp.float32),
            jax.ShapeDtypeStruct((1, G), jnp.float32),
        ),
    )(h2, maskf, xinitf, degf, onehot, onehotT, tvol2)


def kernel(x, edge_index, batch, tvol, p):
    row, col = edge_index[0], edge_index[1]

    zvec = jnp.zeros((_NPAD,), jnp.float32)
    zmat = jnp.zeros((_NPAD, H1), jnp.float32)
    z16 = jnp.zeros((_NPAD, 16), jnp.float32)

    xflat = jnp.zeros((_NPAD,), jnp.float32).at[:N].set(x[:, 0])
    xaggp, mmp, degp = _edge_init_kernel(xflat, row, col, zvec)
    degf = degp[0] + degp[1]

    def v2d(v):
        return v.reshape(_NPAD, 1)

    h, maskc = _gin1_tc(v2d(xflat), v2d(xaggp[0]), v2d(xaggp[1]),
                        v2d(mmp[0]), v2d(mmp[1]), p)

    for i in range(3):
        aggp, mmp = _gin_mask_kernel(h, maskc.reshape(_NPAD), row, col,
                                     zmat, zvec)
        h, maskc = _gin_tc(h, aggp[0], aggp[1], maskc,
                           v2d(mmp[0]), v2d(mmp[1]), i, p)

    sel8 = (jnp.arange(HEADS * H2)[:, None] // H2
            == jnp.arange(HEADS)[None, :])
    Asrc = jnp.where(sel8, p['gat_asrc'].reshape(HEADS * H2)[:, None], 0.0)
    Adst = jnp.where(sel8, p['gat_adst'].reshape(HEADS * H2)[:, None], 0.0)
    hh0, hh1, hh2, hh3, a16, b16 = _gatprep_tc(h, p['gat_W'], Asrc, Adst)
    eaf, denp, mmp = _gat1_kernel(a16, b16, maskc.reshape(_NPAD), row, col,
                                  z16, zvec)
    eaflat = eaf.reshape(E * 16)
    nump = _gat2_kernel(hh0, hh1, hh2, hh3, eaflat, row, col, zmat)
    nums8 = [nump[cnk, sc] for cnk in range(4) for sc in range(_NC)]
    E2 = (jnp.arange(H1)[None, :] // H2
          == jnp.arange(2)[:, None]).astype(jnp.float32)
    h2, maskc = _gatpost_tc(nums8, denp[0], denp[1], maskc,
                            v2d(mmp[0]), v2d(mmp[1]), E2, p)

    onehot = (batch[:, None] == jnp.arange(G)[None, :]).astype(jnp.float32)
    onehot = jnp.zeros((_NPAD, G), jnp.float32).at[:N].set(onehot)
    onehotT = onehot.T
    probs2, cut1 = _niter_tc(
        h2.reshape(1, _NPAD), maskc.reshape(1, _NPAD),
        xflat.reshape(1, _NPAD), degf.reshape(1, _NPAD),
        onehot, onehotT, tvol.reshape(1, G))
    probs = probs2[0, :N]

    bpad = jnp.zeros((_NPAD,), jnp.int32).at[:N].set(batch)
    p16 = jnp.tile(probs2.reshape(_NPAD, 1), (1, 16))
    cutp = _cut_kernel(p16, bpad, row, col)
    cut2 = cutp[0, :, 0] + cutp[1, :, 0]
    expected_cut = cut1[0][:, None] - cut2[:, None]
    return probs, expected_cut
